# baseline (speedup 1.0000x reference)
import functools

import jax
import jax.numpy as jnp
from jax import lax
from jax.experimental import pallas as pl
from jax.experimental.pallas import tpu as pltpu
from jax.experimental.pallas import tpu_sc as plsc

B = 16384
NROWS = 1000000
EMBED = 16
NC = 2
NS = 16
NW = NC * NS
BPW = B // NW
CHUNK = 128
NCHUNK = BPW // CHUNK


def _sc_gather_partial(uflat, uidx, fflat, fidx, ubias, fbias):
  mesh = plsc.VectorSubcoreMesh(core_axis_name="c", subcore_axis_name="s")

  @functools.partial(
      pl.kernel,
      mesh=mesh,
      compiler_params=pltpu.CompilerParams(needs_layout_passes=False),
      out_type=(
          jax.ShapeDtypeStruct((NW * EMBED,), jnp.float32),
          jax.ShapeDtypeStruct((B,), jnp.float32),
      ),
      scratch_types=[
          pltpu.VMEM((NCHUNK, CHUNK), jnp.int32),
          pltpu.VMEM((NCHUNK, CHUNK), jnp.int32),
          pltpu.VMEM((EMBED, CHUNK), jnp.int32),
          pltpu.VMEM((EMBED, CHUNK), jnp.int32),
          pltpu.VMEM((EMBED, CHUNK), jnp.float32),
          pltpu.VMEM((EMBED, CHUNK), jnp.float32),
          pltpu.VMEM((BPW,), jnp.float32),
          pltpu.VMEM((BPW,), jnp.float32),
          pltpu.VMEM((BPW,), jnp.float32),
          pltpu.VMEM((EMBED,), jnp.float32),
          pltpu.SemaphoreType.DMA,
          pltpu.SemaphoreType.DMA,
      ],
  )
  def k(uemb_h, uidx_h, femb_h, fidx_h, ub_h, fb_h,
        part_h, bsum_h,
        uidx_v, fidx_v, uti_v, fti_v, ustr_v, fstr_v,
        ubv, fbv, bs_v, acc_v, sem, bsem):
    wid = lax.axis_index("s") * NC + lax.axis_index("c")
    base = wid * BPW
    pltpu.sync_copy(uidx_h.at[wid], uidx_v)
    pltpu.sync_copy(fidx_h.at[wid], fidx_v)

    bias_copies = []
    for c in range(NCHUNK):
      sl = pl.ds(c * CHUNK, CHUNK)
      bias_copies.append(pltpu.async_copy(ub_h.at[uidx_v.at[c]], ubv.at[sl], bsem))
      bias_copies.append(pltpu.async_copy(fb_h.at[fidx_v.at[c]], fbv.at[sl], bsem))

    zf = jnp.zeros((16,), jnp.float32)
    accs = [zf, zf, zf, zf]
    for c in range(NCHUNK):
      for g in range(CHUNK // 16):
        sl = pl.ds(g * 16, 16)
        uv = uidx_v[c, sl]
        fv = fidx_v[c, sl]
        for e in range(EMBED):
          uti_v[e, sl] = uv + (e * NROWS)
          fti_v[e, sl] = fv + (e * NROWS)
      copies = []
      for e in range(EMBED):
        copies.append(pltpu.async_copy(
            uemb_h.at[uti_v.at[e]], ustr_v.at[e], sem))
        copies.append(pltpu.async_copy(
            femb_h.at[fti_v.at[e]], fstr_v.at[e], sem))
      for cpy in copies:
        cpy.wait()
      for e in range(EMBED):
        for g in range(CHUNK // 16):
          sl = pl.ds(g * 16, 16)
          accs[g % 4] = accs[g % 4] + ustr_v[e, sl] * fstr_v[e, sl]

    acc_v[:] = (accs[0] + accs[1]) + (accs[2] + accs[3])

    for cpy in bias_copies:
      cpy.wait()
    for c in range(NCHUNK):
      for g in range(CHUNK // 16):
        sl = pl.ds(c * CHUNK + g * 16, 16)
        bs_v[sl] = ubv[sl] + fbv[sl]

    pltpu.sync_copy(acc_v, part_h.at[pl.ds(wid * EMBED, EMBED)])
    pltpu.sync_copy(bs_v, bsum_h.at[pl.ds(base, BPW)])

  return k(uflat, uidx, fflat, fidx, ubias, fbias)


def _tc_finish(partials, bsum):
  def body(p_ref, b_ref, o_ref):
    s = jnp.sum(p_ref[:])
    o_ref[:] = 1.0 / (1.0 + jnp.exp(-(b_ref[:] + s)))

  return pl.pallas_call(
      body,
      out_shape=jax.ShapeDtypeStruct((128, 128), jnp.float32),
  )(partials, bsum)


def kernel(inputs, user_embedding, user_bias, food_embedding, food_bias):
  uidx = inputs[:, 0].astype(jnp.int32).reshape(NW, NCHUNK, CHUNK)
  fidx = inputs[:, -1].astype(jnp.int32).reshape(NW, NCHUNK, CHUNK)
  part, bsum = _sc_gather_partial(
      user_embedding.T.reshape(-1), uidx,
      food_embedding.T.reshape(-1), fidx,
      user_bias.reshape(-1), food_bias.reshape(-1))
  out = _tc_finish(part.reshape(4, 128), bsum.reshape(128, 128))
  return out.reshape(B, 1)


# speedup vs baseline: 2.9572x; 2.9572x over previous
import functools

import jax
import jax.numpy as jnp
from jax import lax
from jax.experimental import pallas as pl
from jax.experimental.pallas import tpu as pltpu
from jax.experimental.pallas import tpu_sc as plsc

B = 16384
EMBED = 16
NC = 2
NS = 16
NW = NC * NS
BPW = B // NW
CHUNK = 128
NCHUNK = BPW // CHUNK
NGRP = CHUNK // 16

TCOLS = 512
NSTEP = 1000000 // TCOLS - 1
SPW = NSTEP // NW


def _sc_transpose(uembT, fembT):
  mesh = plsc.VectorSubcoreMesh(core_axis_name="c", subcore_axis_name="s")

  @functools.partial(
      pl.kernel,
      mesh=mesh,
      compiler_params=pltpu.CompilerParams(needs_layout_passes=False),
      out_type=(
          jax.ShapeDtypeStruct((125000, 128), jnp.float32),
          jax.ShapeDtypeStruct((125000, 128), jnp.float32),
      ),
      scratch_types=[
          pltpu.VMEM((EMBED, TCOLS), jnp.float32),
          pltpu.VMEM((EMBED, TCOLS), jnp.float32),
          pltpu.VMEM((TCOLS // 8, 128), jnp.float32),
          pltpu.VMEM((TCOLS // 8, 128), jnp.float32),
          pltpu.VMEM((EMBED, 64), jnp.float32),
          pltpu.VMEM((8, 128), jnp.float32),
          pltpu.SemaphoreType.DMA,
          pltpu.SemaphoreType.DMA,
      ],
  )
  def k(uemb_h, femb_h, uout_h, fout_h,
        uin_v, fin_v, utr_v, ftr_v, tin_v, ttr_v, sem, osem):
    wid = lax.axis_index("s") * NC + lax.axis_index("c")
    lanes = lax.iota(jnp.int32, 16)

    def transpose_block(src_v, dst_v, ncols):
      def tb(i, carry):
        r4 = i * 4
        for kk in range(4):
          rr = r4 + kk
          cols = jnp.broadcast_to(rr, (16,)).astype(jnp.int32)
          row = plsc.load_gather(src_v, [lanes, cols])
          dst_v[rr >> 3, pl.ds(pl.multiple_of((rr & 7) * 16, 16), 16)] = row
        return carry
      lax.fori_loop(0, ncols // 4, tb, 0)

    def step(i, carry):
      s = wid * SPW + i
      col0 = pl.multiple_of(s * TCOLS, TCOLS)
      cu = pltpu.async_copy(uemb_h.at[:, pl.ds(col0, TCOLS)], uin_v, sem)
      cf = pltpu.async_copy(femb_h.at[:, pl.ds(col0, TCOLS)], fin_v, sem)
      cu.wait()
      transpose_block(uin_v, utr_v, TCOLS)
      row0 = pl.multiple_of(s * (TCOLS // 8), TCOLS // 8)
      ou = pltpu.async_copy(utr_v, uout_h.at[pl.ds(row0, TCOLS // 8), :], osem)
      cf.wait()
      transpose_block(fin_v, ftr_v, TCOLS)
      of = pltpu.async_copy(ftr_v, fout_h.at[pl.ds(row0, TCOLS // 8), :], osem)
      ou.wait()
      of.wait()
      return carry

    lax.fori_loop(0, SPW, step, 0)

    @pl.when(wid == 0)
    def _():
      col0 = pl.multiple_of(NSTEP * TCOLS, TCOLS)
      row0 = pl.multiple_of(NSTEP * (TCOLS // 8), TCOLS // 8)
      pltpu.async_copy(uemb_h.at[:, pl.ds(col0, TCOLS)], uin_v, sem).wait()
      transpose_block(uin_v, utr_v, TCOLS)
      ou = pltpu.async_copy(utr_v, uout_h.at[pl.ds(row0, TCOLS // 8), :], osem)
      pltpu.async_copy(femb_h.at[:, pl.ds(col0, TCOLS)], fin_v, sem).wait()
      transpose_block(fin_v, ftr_v, TCOLS)
      pltpu.async_copy(ftr_v, fout_h.at[pl.ds(row0, TCOLS // 8), :], osem).wait()
      ou.wait()

    @pl.when(wid == 1)
    def _():
      pltpu.async_copy(uemb_h.at[:, pl.ds(999936, 64)], tin_v, sem).wait()
      transpose_block(tin_v, ttr_v, 64)
      ot = pltpu.async_copy(ttr_v, uout_h.at[pl.ds(124992, 8), :], osem)
      pltpu.async_copy(femb_h.at[:, pl.ds(999936, 64)], tin_v, sem).wait()
      ot.wait()
      transpose_block(tin_v, ttr_v, 64)
      pltpu.async_copy(ttr_v, fout_h.at[pl.ds(124992, 8), :], osem).wait()

  return k(uembT, fembT)


def _sc_gather_partial(uemb2, uidx, femb2, fidx, ubias, fbias):
  mesh = plsc.VectorSubcoreMesh(core_axis_name="c", subcore_axis_name="s")

  @functools.partial(
      pl.kernel,
      mesh=mesh,
      compiler_params=pltpu.CompilerParams(needs_layout_passes=False),
      out_type=(
          jax.ShapeDtypeStruct((NW * EMBED,), jnp.float32),
          jax.ShapeDtypeStruct((B,), jnp.float32),
      ),
      scratch_types=[
          pltpu.VMEM((NCHUNK, CHUNK), jnp.int32),
          pltpu.VMEM((NCHUNK, CHUNK), jnp.int32),
          pltpu.VMEM((NCHUNK, CHUNK), jnp.int32),
          pltpu.VMEM((NCHUNK, CHUNK), jnp.int32),
          pltpu.VMEM((CHUNK, 128), jnp.float32),
          pltpu.VMEM((CHUNK, 128), jnp.float32),
          pltpu.VMEM((BPW,), jnp.float32),
          pltpu.VMEM((BPW,), jnp.float32),
          pltpu.VMEM((BPW,), jnp.float32),
          pltpu.VMEM((EMBED,), jnp.float32),
          pltpu.SemaphoreType.DMA,
          pltpu.SemaphoreType.DMA,
      ],
  )
  def k(uemb_h, uidx_h, femb_h, fidx_h, ub_h, fb_h,
        part_h, bsum_h,
        uidx_v, fidx_v, ublk_v, fblk_v, urows_v, frows_v,
        ubv, fbv, bs_v, acc_v, sem, bsem):
    wid = lax.axis_index("s") * NC + lax.axis_index("c")
    base = wid * BPW
    pltpu.sync_copy(uidx_h.at[wid], uidx_v)
    pltpu.sync_copy(fidx_h.at[wid], fidx_v)

    for c in range(NCHUNK):
      for g in range(NGRP):
        sl = pl.ds(g * 16, 16)
        ublk_v[c, sl] = uidx_v[c, sl] >> 3
        fblk_v[c, sl] = fidx_v[c, sl] >> 3

    bias_copies = []
    for c in range(NCHUNK):
      sl = pl.ds(c * CHUNK, CHUNK)
      bias_copies.append(pltpu.async_copy(ub_h.at[uidx_v.at[c]], ubv.at[sl], bsem))
      bias_copies.append(pltpu.async_copy(fb_h.at[fidx_v.at[c]], fbv.at[sl], bsem))

    lanes = lax.iota(jnp.int32, 16)
    zero = jnp.zeros((EMBED,), jnp.float32)
    accs = [zero, zero, zero, zero]
    for c in range(NCHUNK):
      cu = pltpu.async_copy(uemb_h.at[ublk_v.at[c]], urows_v, sem)
      cf = pltpu.async_copy(femb_h.at[fblk_v.at[c]], frows_v, sem)
      cu.wait()
      cf.wait()
      for g in range(NGRP):
        sl = pl.ds(g * 16, 16)
        uidx16 = uidx_v[c, sl]
        fidx16 = fidx_v[c, sl]
        ucol0 = (uidx16 & 7) * 16
        fcol0 = (fidx16 & 7) * 16
        rows = lanes + (g * 16)
        for e in range(EMBED):
          u_e = plsc.load_gather(urows_v, [rows, ucol0 + e])
          f_e = plsc.load_gather(frows_v, [rows, fcol0 + e])
          accs[e % 4] = accs[e % 4] + u_e * f_e

    for cpy in bias_copies:
      cpy.wait()

    for c in range(NCHUNK):
      for g in range(NGRP):
        sl = pl.ds(c * CHUNK + g * 16, 16)
        bs_v[sl] = ubv[sl] + fbv[sl]

    acc_v[:] = (accs[0] + accs[1]) + (accs[2] + accs[3])
    pltpu.sync_copy(acc_v, part_h.at[pl.ds(wid * EMBED, EMBED)])
    pltpu.sync_copy(bs_v, bsum_h.at[pl.ds(base, BPW)])

  return k(uemb2, uidx, femb2, fidx, ubias, fbias)


def _tc_finish(partials, bsum):
  def body(p_ref, b_ref, o_ref):
    s = jnp.sum(p_ref[:])
    o_ref[:] = 1.0 / (1.0 + jnp.exp(-(b_ref[:] + s)))

  return pl.pallas_call(
      body,
      out_shape=jax.ShapeDtypeStruct((128, 128), jnp.float32),
  )(partials, bsum)


def kernel(inputs, user_embedding, user_bias, food_embedding, food_bias):
  uidx = inputs[:, 0].astype(jnp.int32).reshape(NW, NCHUNK, CHUNK)
  fidx = inputs[:, -1].astype(jnp.int32).reshape(NW, NCHUNK, CHUNK)
  uemb2, femb2 = _sc_transpose(user_embedding.T, food_embedding.T)
  part, bsum = _sc_gather_partial(
      uemb2, uidx, femb2, fidx,
      user_bias.reshape(-1), food_bias.reshape(-1))
  out = _tc_finish(part.reshape(4, 128), bsum.reshape(128, 128))
  return out.reshape(B, 1)


# speedup vs baseline: 2.9603x; 1.0011x over previous
import functools

import jax
import jax.numpy as jnp
from jax import lax
from jax.experimental import pallas as pl
from jax.experimental.pallas import tpu as pltpu
from jax.experimental.pallas import tpu_sc as plsc

B = 16384
EMBED = 16
NC = 2
NS = 16
NW = NC * NS
BPW = B // NW
CHUNK = 128
NCHUNK = BPW // CHUNK
NGRP = CHUNK // 16

TCOLS = 512
NSTEP = 1000000 // TCOLS - 1
SPW = NSTEP // NW


def _sc_transpose(uembT, fembT):
  mesh = plsc.VectorSubcoreMesh(core_axis_name="c", subcore_axis_name="s")

  @functools.partial(
      pl.kernel,
      mesh=mesh,
      compiler_params=pltpu.CompilerParams(needs_layout_passes=False),
      out_type=(
          jax.ShapeDtypeStruct((125000, 128), jnp.float32),
          jax.ShapeDtypeStruct((125000, 128), jnp.float32),
      ),
      scratch_types=[
          pltpu.VMEM((EMBED, TCOLS), jnp.float32),
          pltpu.VMEM((EMBED, TCOLS), jnp.float32),
          pltpu.VMEM((TCOLS // 8, 128), jnp.float32),
          pltpu.VMEM((TCOLS // 8, 128), jnp.float32),
          pltpu.VMEM((EMBED, 64), jnp.float32),
          pltpu.VMEM((8, 128), jnp.float32),
          pltpu.SemaphoreType.DMA,
          pltpu.SemaphoreType.DMA,
      ],
  )
  def k(uemb_h, femb_h, uout_h, fout_h,
        uin_v, fin_v, utr_v, ftr_v, tin_v, ttr_v, sem, osem):
    wid = lax.axis_index("s") * NC + lax.axis_index("c")
    lanes = lax.iota(jnp.int32, 16)

    def transpose_block(src_v, dst_v, ncols):
      def tb(i, carry):
        base = jnp.broadcast_to(i * 8, (16,)).astype(jnp.int32)
        for kk in range(8):
          row = plsc.load_gather(src_v, [lanes, base + kk])
          dst_v[i, pl.ds(kk * 16, 16)] = row
        return carry
      lax.fori_loop(0, ncols // 8, tb, 0)

    def step(i, carry):
      s = wid * SPW + i
      col0 = pl.multiple_of(s * TCOLS, TCOLS)
      cu = pltpu.async_copy(uemb_h.at[:, pl.ds(col0, TCOLS)], uin_v, sem)
      cf = pltpu.async_copy(femb_h.at[:, pl.ds(col0, TCOLS)], fin_v, sem)
      cu.wait()
      transpose_block(uin_v, utr_v, TCOLS)
      row0 = pl.multiple_of(s * (TCOLS // 8), TCOLS // 8)
      ou = pltpu.async_copy(utr_v, uout_h.at[pl.ds(row0, TCOLS // 8), :], osem)
      cf.wait()
      transpose_block(fin_v, ftr_v, TCOLS)
      of = pltpu.async_copy(ftr_v, fout_h.at[pl.ds(row0, TCOLS // 8), :], osem)
      ou.wait()
      of.wait()
      return carry

    lax.fori_loop(0, SPW, step, 0)

    @pl.when(wid == 0)
    def _():
      col0 = pl.multiple_of(NSTEP * TCOLS, TCOLS)
      row0 = pl.multiple_of(NSTEP * (TCOLS // 8), TCOLS // 8)
      pltpu.async_copy(uemb_h.at[:, pl.ds(col0, TCOLS)], uin_v, sem).wait()
      transpose_block(uin_v, utr_v, TCOLS)
      ou = pltpu.async_copy(utr_v, uout_h.at[pl.ds(row0, TCOLS // 8), :], osem)
      pltpu.async_copy(femb_h.at[:, pl.ds(col0, TCOLS)], fin_v, sem).wait()
      transpose_block(fin_v, ftr_v, TCOLS)
      pltpu.async_copy(ftr_v, fout_h.at[pl.ds(row0, TCOLS // 8), :], osem).wait()
      ou.wait()

    @pl.when(wid == 1)
    def _():
      pltpu.async_copy(uemb_h.at[:, pl.ds(999936, 64)], tin_v, sem).wait()
      transpose_block(tin_v, ttr_v, 64)
      ot = pltpu.async_copy(ttr_v, uout_h.at[pl.ds(124992, 8), :], osem)
      pltpu.async_copy(femb_h.at[:, pl.ds(999936, 64)], tin_v, sem).wait()
      ot.wait()
      transpose_block(tin_v, ttr_v, 64)
      pltpu.async_copy(ttr_v, fout_h.at[pl.ds(124992, 8), :], osem).wait()

  return k(uembT, fembT)


def _sc_gather_partial(uemb2, uidx, femb2, fidx, ubias, fbias):
  mesh = plsc.VectorSubcoreMesh(core_axis_name="c", subcore_axis_name="s")

  @functools.partial(
      pl.kernel,
      mesh=mesh,
      compiler_params=pltpu.CompilerParams(needs_layout_passes=False),
      out_type=(
          jax.ShapeDtypeStruct((NW * EMBED,), jnp.float32),
          jax.ShapeDtypeStruct((B,), jnp.float32),
      ),
      scratch_types=[
          pltpu.VMEM((NCHUNK, CHUNK), jnp.int32),
          pltpu.VMEM((NCHUNK, CHUNK), jnp.int32),
          pltpu.VMEM((NCHUNK, CHUNK), jnp.int32),
          pltpu.VMEM((NCHUNK, CHUNK), jnp.int32),
          pltpu.VMEM((CHUNK, 128), jnp.float32),
          pltpu.VMEM((CHUNK, 128), jnp.float32),
          pltpu.VMEM((BPW,), jnp.float32),
          pltpu.VMEM((BPW,), jnp.float32),
          pltpu.VMEM((BPW,), jnp.float32),
          pltpu.VMEM((EMBED,), jnp.float32),
          pltpu.SemaphoreType.DMA,
          pltpu.SemaphoreType.DMA,
      ],
  )
  def k(uemb_h, uidx_h, femb_h, fidx_h, ub_h, fb_h,
        part_h, bsum_h,
        uidx_v, fidx_v, ublk_v, fblk_v, urows_v, frows_v,
        ubv, fbv, bs_v, acc_v, sem, bsem):
    wid = lax.axis_index("s") * NC + lax.axis_index("c")
    base = wid * BPW
    pltpu.sync_copy(uidx_h.at[wid], uidx_v)
    pltpu.sync_copy(fidx_h.at[wid], fidx_v)

    for c in range(NCHUNK):
      for g in range(NGRP):
        sl = pl.ds(g * 16, 16)
        ublk_v[c, sl] = uidx_v[c, sl] >> 3
        fblk_v[c, sl] = fidx_v[c, sl] >> 3

    bias_copies = []
    for c in range(NCHUNK):
      sl = pl.ds(c * CHUNK, CHUNK)
      bias_copies.append(pltpu.async_copy(ub_h.at[uidx_v.at[c]], ubv.at[sl], bsem))
      bias_copies.append(pltpu.async_copy(fb_h.at[fidx_v.at[c]], fbv.at[sl], bsem))

    lanes = lax.iota(jnp.int32, 16)
    zero = jnp.zeros((EMBED,), jnp.float32)
    accs = [zero, zero, zero, zero]
    for c in range(NCHUNK):
      cu = pltpu.async_copy(uemb_h.at[ublk_v.at[c]], urows_v, sem)
      cf = pltpu.async_copy(femb_h.at[fblk_v.at[c]], frows_v, sem)
      cu.wait()
      cf.wait()
      for g in range(NGRP):
        sl = pl.ds(g * 16, 16)
        uidx16 = uidx_v[c, sl]
        fidx16 = fidx_v[c, sl]
        ucol0 = (uidx16 & 7) * 16
        fcol0 = (fidx16 & 7) * 16
        rows = lanes + (g * 16)
        for e in range(EMBED):
          u_e = plsc.load_gather(urows_v, [rows, ucol0 + e])
          f_e = plsc.load_gather(frows_v, [rows, fcol0 + e])
          accs[e % 4] = accs[e % 4] + u_e * f_e

    for cpy in bias_copies:
      cpy.wait()

    for c in range(NCHUNK):
      for g in range(NGRP):
        sl = pl.ds(c * CHUNK + g * 16, 16)
        bs_v[sl] = ubv[sl] + fbv[sl]

    acc_v[:] = (accs[0] + accs[1]) + (accs[2] + accs[3])
    pltpu.sync_copy(acc_v, part_h.at[pl.ds(wid * EMBED, EMBED)])
    pltpu.sync_copy(bs_v, bsum_h.at[pl.ds(base, BPW)])

  return k(uemb2, uidx, femb2, fidx, ubias, fbias)


def _tc_finish(partials, bsum):
  def body(p_ref, b_ref, o_ref):
    s = jnp.sum(p_ref[:])
    o_ref[:] = 1.0 / (1.0 + jnp.exp(-(b_ref[:] + s)))

  return pl.pallas_call(
      body,
      out_shape=jax.ShapeDtypeStruct((128, 128), jnp.float32),
  )(partials, bsum)


def kernel(inputs, user_embedding, user_bias, food_embedding, food_bias):
  uidx = inputs[:, 0].astype(jnp.int32).reshape(NW, NCHUNK, CHUNK)
  fidx = inputs[:, -1].astype(jnp.int32).reshape(NW, NCHUNK, CHUNK)
  uemb2, femb2 = _sc_transpose(user_embedding.T, food_embedding.T)
  part, bsum = _sc_gather_partial(
      uemb2, uidx, femb2, fidx,
      user_bias.reshape(-1), food_bias.reshape(-1))
  out = _tc_finish(part.reshape(4, 128), bsum.reshape(128, 128))
  return out.reshape(B, 1)


# speedup vs baseline: 3.2466x; 1.0967x over previous
import functools

import jax
import jax.numpy as jnp
from jax import lax
from jax.experimental import pallas as pl
from jax.experimental.pallas import tpu as pltpu
from jax.experimental.pallas import tpu_sc as plsc

B = 16384
EMBED = 16
NC = 2
NS = 16
NW = NC * NS
BPW = B // NW
CHUNK = 128
NCHUNK = BPW // CHUNK
NGRP = CHUNK // 16


def _sc_gather_partial(uemb2, uidx, femb2, fidx, ubias, fbias):
  mesh = plsc.VectorSubcoreMesh(core_axis_name="c", subcore_axis_name="s")

  @functools.partial(
      pl.kernel,
      mesh=mesh,
      compiler_params=pltpu.CompilerParams(needs_layout_passes=False),
      out_type=(
          jax.ShapeDtypeStruct((NW * EMBED,), jnp.float32),
          jax.ShapeDtypeStruct((B,), jnp.float32),
      ),
      scratch_types=[
          pltpu.VMEM((NCHUNK, CHUNK), jnp.int32),
          pltpu.VMEM((NCHUNK, CHUNK), jnp.int32),
          pltpu.VMEM((NCHUNK, CHUNK), jnp.int32),
          pltpu.VMEM((NCHUNK, CHUNK), jnp.int32),
          pltpu.VMEM((CHUNK, 128), jnp.float32),
          pltpu.VMEM((CHUNK, 128), jnp.float32),
          pltpu.VMEM((BPW,), jnp.float32),
          pltpu.VMEM((BPW,), jnp.float32),
          pltpu.VMEM((BPW,), jnp.float32),
          pltpu.VMEM((EMBED,), jnp.float32),
          pltpu.SemaphoreType.DMA,
          pltpu.SemaphoreType.DMA,
      ],
  )
  def k(uemb_h, uidx_h, femb_h, fidx_h, ub_h, fb_h,
        part_h, bsum_h,
        uidx_v, fidx_v, ublk_v, fblk_v, urows_v, frows_v,
        ubv, fbv, bs_v, acc_v, sem, bsem):
    wid = lax.axis_index("s") * NC + lax.axis_index("c")
    base = wid * BPW
    pltpu.sync_copy(uidx_h.at[wid], uidx_v)
    pltpu.sync_copy(fidx_h.at[wid], fidx_v)

    for c in range(NCHUNK):
      for g in range(NGRP):
        sl = pl.ds(g * 16, 16)
        ublk_v[c, sl] = uidx_v[c, sl] >> 3
        fblk_v[c, sl] = fidx_v[c, sl] >> 3

    bias_copies = []
    for c in range(NCHUNK):
      sl = pl.ds(c * CHUNK, CHUNK)
      bias_copies.append(pltpu.async_copy(ub_h.at[uidx_v.at[c]], ubv.at[sl], bsem))
      bias_copies.append(pltpu.async_copy(fb_h.at[fidx_v.at[c]], fbv.at[sl], bsem))

    lanes = lax.iota(jnp.int32, 16)
    zero = jnp.zeros((EMBED,), jnp.float32)
    accs = [zero, zero, zero, zero]
    for c in range(NCHUNK):
      cu = pltpu.async_copy(uemb_h.at[ublk_v.at[c]], urows_v, sem)
      cf = pltpu.async_copy(femb_h.at[fblk_v.at[c]], frows_v, sem)
      cu.wait()
      cf.wait()
      for g in range(NGRP):
        sl = pl.ds(g * 16, 16)
        uidx16 = uidx_v[c, sl]
        fidx16 = fidx_v[c, sl]
        ucol0 = (uidx16 & 7) * 16
        fcol0 = (fidx16 & 7) * 16
        rows = lanes + (g * 16)
        for e in range(EMBED):
          u_e = plsc.load_gather(urows_v, [rows, ucol0 + e])
          f_e = plsc.load_gather(frows_v, [rows, fcol0 + e])
          accs[e % 4] = accs[e % 4] + u_e * f_e

    for cpy in bias_copies:
      cpy.wait()

    for c in range(NCHUNK):
      for g in range(NGRP):
        sl = pl.ds(c * CHUNK + g * 16, 16)
        bs_v[sl] = ubv[sl] + fbv[sl]

    acc_v[:] = (accs[0] + accs[1]) + (accs[2] + accs[3])
    pltpu.sync_copy(acc_v, part_h.at[pl.ds(wid * EMBED, EMBED)])
    pltpu.sync_copy(bs_v, bsum_h.at[pl.ds(base, BPW)])

  return k(uemb2, uidx, femb2, fidx, ubias, fbias)


def _tc_finish(partials, bsum):
  def body(p_ref, b_ref, o_ref):
    s = jnp.sum(p_ref[:])
    o_ref[:] = 1.0 / (1.0 + jnp.exp(-(b_ref[:] + s)))

  return pl.pallas_call(
      body,
      out_shape=jax.ShapeDtypeStruct((128, 128), jnp.float32),
  )(partials, bsum)


def kernel(inputs, user_embedding, user_bias, food_embedding, food_bias):
  uidx = inputs[:, 0].astype(jnp.int32).reshape(NW, NCHUNK, CHUNK)
  fidx = inputs[:, -1].astype(jnp.int32).reshape(NW, NCHUNK, CHUNK)
  uemb2 = user_embedding.reshape(125000, 128)
  femb2 = food_embedding.reshape(125000, 128)
  part, bsum = _sc_gather_partial(
      uemb2, uidx, femb2, fidx,
      user_bias.reshape(-1), food_bias.reshape(-1))
  out = _tc_finish(part.reshape(4, 128), bsum.reshape(128, 128))
  return out.reshape(B, 1)
